# baseline (speedup 1.0000x reference)
import functools

import jax
import jax.numpy as jnp
from jax import lax
from jax.experimental import pallas as pl
from jax.experimental.pallas import tpu as pltpu
from jax.experimental.pallas import tpu_sc as plsc

VOCAB_ROWS = 1000008
DIM = 300
B_ROWS = 4096
B_COLS = 200
NUM_IDX = B_ROWS * B_COLS

NC = 2
NS = 16
NW = NC * NS
PER_W = NUM_IDX // NW
CHUNK = 128
NCHUNKS = PER_W // CHUNK
NBUF = 2


def _body(table_hbm, idx_hbm, out_hbm, idx_v, rows_v, gsem0, gsem1,
          wsem0, wsem1):
    wid = lax.axis_index("s") * NC + lax.axis_index("c")
    base = wid * PER_W
    pltpu.sync_copy(idx_hbm.at[pl.ds(base, PER_W)], idx_v)

    gsems = (gsem0, gsem1)
    wsems = (wsem0, wsem1)

    def start_gather(c, b):
        pltpu.async_copy(
            table_hbm.at[idx_v.at[pl.ds(c * CHUNK, CHUNK)]],
            rows_v.at[b],
            gsems[b],
        )

    def start_write(c, b):
        pltpu.async_copy(
            rows_v.at[b],
            out_hbm.at[pl.ds(base + c * CHUNK, CHUNK)],
            wsems[b],
        )

    for b in range(NBUF):
        start_gather(b, b)

    def outer(g, carry):
        for b in range(NBUF):
            c = g * NBUF + b
            pltpu.make_async_copy(
                table_hbm.at[idx_v.at[pl.ds(c * CHUNK, CHUNK)]],
                rows_v.at[b],
                gsems[b],
            ).wait()
            start_write(c, b)
            pltpu.make_async_copy(
                rows_v.at[b],
                out_hbm.at[pl.ds(base + c * CHUNK, CHUNK)],
                wsems[b],
            ).wait()

            @pl.when(c + NBUF < NCHUNKS)
            def _():
                start_gather(c + NBUF, b)
        return carry

    lax.fori_loop(0, NCHUNKS // NBUF, outer, 0)


@functools.partial(jax.jit, static_argnums=())
def _gather_rows(table, flat_idx):
    mesh = plsc.VectorSubcoreMesh(core_axis_name="c", subcore_axis_name="s")
    k = functools.partial(
        pl.kernel,
        mesh=mesh,
        out_type=jax.ShapeDtypeStruct((NUM_IDX, DIM), jnp.float32),
        scratch_types=[
            pltpu.VMEM((PER_W,), jnp.int32),
            pltpu.VMEM((NBUF, CHUNK, DIM), jnp.float32),
            pltpu.SemaphoreType.DMA,
            pltpu.SemaphoreType.DMA,
            pltpu.SemaphoreType.DMA,
            pltpu.SemaphoreType.DMA,
        ],
        compiler_params=pltpu.CompilerParams(use_tc_tiling_on_sc=False),
    )(_body)
    return k(table, flat_idx)


def kernel(table, idxes):
    flat_idx = idxes.reshape(NUM_IDX).astype(jnp.int32)
    out = _gather_rows(table, flat_idx)
    return out.reshape(B_ROWS, B_COLS, DIM)


# speedup vs baseline: 2.3585x; 2.3585x over previous
import functools

import jax
import jax.numpy as jnp
from jax import lax
from jax.experimental import pallas as pl
from jax.experimental.pallas import tpu as pltpu
from jax.experimental.pallas import tpu_sc as plsc

VOCAB = 1000008
DIM = 300
B_ROWS = 4096
B_COLS = 200
NUM_IDX = B_ROWS * B_COLS

NC = 2
NS = 16
C_PER_SC = DIM // NC
J_PER_TILE = NUM_IDX // NS
GCHUNK = 6400
N_G = J_PER_TILE // GCHUNK

CHUNK = 7808
N_CHUNKS = 8
PER_TILE_LOAD = CHUNK * N_CHUNKS
MAIN_N = NS * PER_TILE_LOAD
TAIL_A = 512
TAIL_B = VOCAB - MAIN_N - TAIL_A
TAIL_B_OFF = MAIN_N + TAIL_A


def _body(table_hbm, idx_hbm, tail_hbm, out_hbm,
          img, tail_sp, idx_v, bufA, bufB, cbuf, tailrow,
          isem0, isem1, fsem, gsem0, gsem1, wsem0, wsem1):
    sc = lax.axis_index("c")
    tid = lax.axis_index("s")
    cbase = sc * C_PER_SC
    jbase = tid * J_PER_TILE
    lbase = tid * PER_TILE_LOAD

    pltpu.sync_copy(idx_hbm.at[pl.ds(jbase, J_PER_TILE)], idx_v)

    @pl.when(tid == 0)
    def _():
        pltpu.sync_copy(tail_hbm, tail_sp)

    bufs = (bufA, bufB)
    isems = (isem0, isem1)
    gsems = (gsem0, gsem1)
    wsems = (wsem0, wsem1)

    def ind_desc(k):
        cref = cbuf.at[pl.ds(0, 1)]
        return pltpu.make_async_copy(
            table_hbm.at[cref, pl.ds(lbase + k * CHUNK, CHUNK)],
            bufs[k % 2],
            isems[k % 2],
        )

    def fwd_desc(k):
        return pltpu.make_async_copy(
            bufs[k % 2].at[0],
            img.at[pl.ds(lbase + k * CHUNK, CHUNK)],
            fsem,
        )

    def indA_desc():
        cref = cbuf.at[pl.ds(0, 1)]
        return pltpu.make_async_copy(
            table_hbm.at[cref, pl.ds(MAIN_N, TAIL_A)],
            bufA.at[:, pl.ds(0, TAIL_A)],
            isems[0],
        )

    tailA_fwd = pltpu.make_async_copy(
        bufA.at[0, pl.ds(0, TAIL_A)], img.at[pl.ds(MAIN_N, TAIL_A)], fsem)
    tailB_fwd = pltpu.make_async_copy(
        tailrow, img.at[pl.ds(TAIL_B_OFF, TAIL_B)], fsem)

    def start_load(c):
        cbuf[...] = jnp.full((16,), c, jnp.int32)
        ind_desc(0).start()
        ind_desc(1).start()
        for k in range(N_CHUNKS):
            ind_desc(k).wait()
            fwd_desc(k).start()
            if k + 2 < N_CHUNKS:
                fwd_desc(k).wait()
                ind_desc(k + 2).start()

        @pl.when(tid == 15)
        def _():
            fwd_desc(N_CHUNKS - 2).wait()
            ia = indA_desc()
            ia.start()
            ia.wait()
            tailA_fwd.start()

        @pl.when(tid == 0)
        def _():
            tb = pltpu.make_async_copy(
                tail_sp.at[pl.ds(c * TAIL_B, TAIL_B)], tailrow, isem1)
            tb.start()
            tb.wait()
            tailB_fwd.start()

    def wait_load():
        @pl.when(tid != 15)
        def _():
            fwd_desc(N_CHUNKS - 2).wait()
        fwd_desc(N_CHUNKS - 1).wait()

        @pl.when(tid == 15)
        def _():
            tailA_fwd.wait()

        @pl.when(tid == 0)
        def _():
            tailB_fwd.wait()

    def g_desc(b):
        return pltpu.make_async_copy(
            img.at[idx_v.at[pl.ds(b * GCHUNK, GCHUNK)]],
            bufs[b % 2].at[0, pl.ds(0, GCHUNK)],
            gsems[b % 2],
        )

    def w_desc(c, b):
        return pltpu.make_async_copy(
            bufs[b % 2].at[0, pl.ds(0, GCHUNK)],
            out_hbm.at[pl.ds(c * NUM_IDX + jbase + b * GCHUNK, GCHUNK)],
            wsems[b % 2],
        )

    def iter_body(i, carry):
        c = cbase + i
        wait_load()
        plsc.subcore_barrier()
        g_desc(0).start()
        g_desc(1).start()
        for b in range(N_G):
            g_desc(b).wait()
            w_desc(c, b).start()
            w_desc(c, b).wait()
            if b + 2 < N_G:
                g_desc(b + 2).start()
        plsc.subcore_barrier()

        @pl.when(i + 1 < C_PER_SC)
        def _():
            start_load(c + 1)

        return carry

    start_load(cbase)
    lax.fori_loop(0, C_PER_SC, iter_body, 0)


def _gather_t(table_t, idx_flat, tail_1d):
    mesh = plsc.VectorSubcoreMesh(core_axis_name="c", subcore_axis_name="s")
    k = functools.partial(
        pl.kernel,
        mesh=mesh,
        out_type=jax.ShapeDtypeStruct((DIM * NUM_IDX,), jnp.float32),
        scratch_types=[
            pltpu.VMEM_SHARED((VOCAB,), jnp.float32),
            pltpu.VMEM_SHARED((DIM * TAIL_B,), jnp.float32),
            pltpu.VMEM((J_PER_TILE,), jnp.int32),
            pltpu.VMEM((1, CHUNK), jnp.float32),
            pltpu.VMEM((1, CHUNK), jnp.float32),
            pltpu.VMEM((16,), jnp.int32),
            pltpu.VMEM((TAIL_B,), jnp.float32),
            pltpu.SemaphoreType.DMA,
            pltpu.SemaphoreType.DMA,
            pltpu.SemaphoreType.DMA,
            pltpu.SemaphoreType.DMA,
            pltpu.SemaphoreType.DMA,
            pltpu.SemaphoreType.DMA,
            pltpu.SemaphoreType.DMA,
        ],
    )(_body)
    return k(table_t, idx_flat, tail_1d)


def kernel(table, idxes):
    table_t = jnp.transpose(table)
    idx_flat = jnp.transpose(idxes).reshape(NUM_IDX).astype(jnp.int32)
    tail_1d = jnp.transpose(table[TAIL_B_OFF:, :]).reshape(DIM * TAIL_B)
    out_flat = _gather_t(table_t, idx_flat, tail_1d)
    out3 = out_flat.reshape(DIM, B_COLS, B_ROWS)
    return jnp.transpose(out3, (2, 1, 0))


# speedup vs baseline: 2.4340x; 1.0320x over previous
import functools

import jax
import jax.numpy as jnp
from jax import lax
from jax.experimental import pallas as pl
from jax.experimental.pallas import tpu as pltpu
from jax.experimental.pallas import tpu_sc as plsc

VOCAB = 1000008
DIM = 300
B_ROWS = 4096
B_COLS = 200
NUM_IDX = B_ROWS * B_COLS

NC = 2
NS = 16
C_PER_SC = DIM // NC
J_PER_TILE = NUM_IDX // NS
GCHUNK = 3200
N_G = J_PER_TILE // GCHUNK
GSLOT_OFF = 3968

CHUNK = 7808
N_CHUNKS = 8
PER_TILE_LOAD = CHUNK * N_CHUNKS
MAIN_N = NS * PER_TILE_LOAD
TAIL_A = 512
TAIL_B = VOCAB - MAIN_N - TAIL_A
TAIL_B_OFF = MAIN_N + TAIL_A


def _body(table_hbm, idx_hbm, tail_hbm, out_hbm,
          img, tail_sp, idx_v, bufA, bufB, cbuf, tailrow,
          isem0, isem1, fsem, gsem0, gsem1, gsem2, gsem3,
          wsem0, wsem1, wsem2, wsem3):
    sc = lax.axis_index("c")
    tid = lax.axis_index("s")
    cbase = sc * C_PER_SC
    jbase = tid * J_PER_TILE
    lbase = tid * PER_TILE_LOAD

    pltpu.sync_copy(idx_hbm.at[pl.ds(jbase, J_PER_TILE)], idx_v)

    @pl.when(tid == 0)
    def _():
        pltpu.sync_copy(tail_hbm, tail_sp)

    bufs = (bufA, bufB)
    isems = (isem0, isem1)
    gsems = (gsem0, gsem1, gsem2, gsem3)
    wsems = (wsem0, wsem1, wsem2, wsem3)

    def gslot(q):
        return bufs[q % 2].at[0, pl.ds((q // 2) * GSLOT_OFF, GCHUNK)]

    def ind_desc(k):
        cref = cbuf.at[pl.ds(0, 1)]
        return pltpu.make_async_copy(
            table_hbm.at[cref, pl.ds(lbase + k * CHUNK, CHUNK)],
            bufs[k % 2],
            isems[k % 2],
        )

    def fwd_desc(k):
        return pltpu.make_async_copy(
            bufs[k % 2].at[0],
            img.at[pl.ds(lbase + k * CHUNK, CHUNK)],
            fsem,
        )

    def indA_desc():
        cref = cbuf.at[pl.ds(0, 1)]
        return pltpu.make_async_copy(
            table_hbm.at[cref, pl.ds(MAIN_N, TAIL_A)],
            bufA.at[:, pl.ds(0, TAIL_A)],
            isems[0],
        )

    tailA_fwd = pltpu.make_async_copy(
        bufA.at[0, pl.ds(0, TAIL_A)], img.at[pl.ds(MAIN_N, TAIL_A)], fsem)
    tailB_fwd = pltpu.make_async_copy(
        tailrow, img.at[pl.ds(TAIL_B_OFF, TAIL_B)], fsem)

    def start_load(c):
        cbuf[...] = jnp.full((16,), c, jnp.int32)
        ind_desc(0).start()
        ind_desc(1).start()
        for k in range(N_CHUNKS):
            ind_desc(k).wait()
            fwd_desc(k).start()
            if k + 2 < N_CHUNKS:
                fwd_desc(k).wait()
                ind_desc(k + 2).start()

        @pl.when(tid == 15)
        def _():
            fwd_desc(N_CHUNKS - 2).wait()
            ia = indA_desc()
            ia.start()
            ia.wait()
            tailA_fwd.start()

        @pl.when(tid == 0)
        def _():
            tb = pltpu.make_async_copy(
                tail_sp.at[pl.ds(c * TAIL_B, TAIL_B)], tailrow, isem1)
            tb.start()
            tb.wait()
            tailB_fwd.start()

    def wait_load():
        @pl.when(tid != 15)
        def _():
            fwd_desc(N_CHUNKS - 2).wait()
        fwd_desc(N_CHUNKS - 1).wait()

        @pl.when(tid == 15)
        def _():
            tailA_fwd.wait()

        @pl.when(tid == 0)
        def _():
            tailB_fwd.wait()

    def g_desc(b):
        return pltpu.make_async_copy(
            img.at[idx_v.at[pl.ds(b * GCHUNK, GCHUNK)]],
            gslot(b % 4),
            gsems[b % 4],
        )

    def w_desc(c, b):
        return pltpu.make_async_copy(
            gslot(b % 4),
            out_hbm.at[pl.ds(c * NUM_IDX + jbase + b * GCHUNK, GCHUNK)],
            wsems[b % 4],
        )

    def iter_body(i, carry):
        c = cbase + i
        wait_load()
        plsc.subcore_barrier()
        for q in range(4):
            g_desc(q).start()
        for b in range(N_G):
            g_desc(b).wait()
            w_desc(c, b).start()
            if b + 4 < N_G:
                w_desc(c, b).wait()
                g_desc(b + 4).start()
        for b in range(N_G - 4, N_G):
            w_desc(c, b).wait()
        plsc.subcore_barrier()

        @pl.when(i + 1 < C_PER_SC)
        def _():
            start_load(c + 1)

        return carry

    start_load(cbase)
    lax.fori_loop(0, C_PER_SC, iter_body, 0)


def _gather_t(table_t, idx_flat, tail_1d):
    mesh = plsc.VectorSubcoreMesh(core_axis_name="c", subcore_axis_name="s")
    k = functools.partial(
        pl.kernel,
        mesh=mesh,
        out_type=jax.ShapeDtypeStruct((DIM * NUM_IDX,), jnp.float32),
        scratch_types=[
            pltpu.VMEM_SHARED((VOCAB,), jnp.float32),
            pltpu.VMEM_SHARED((DIM * TAIL_B,), jnp.float32),
            pltpu.VMEM((J_PER_TILE,), jnp.int32),
            pltpu.VMEM((1, CHUNK), jnp.float32),
            pltpu.VMEM((1, CHUNK), jnp.float32),
            pltpu.VMEM((16,), jnp.int32),
            pltpu.VMEM((TAIL_B,), jnp.float32),
            pltpu.SemaphoreType.DMA,
            pltpu.SemaphoreType.DMA,
            pltpu.SemaphoreType.DMA,
            pltpu.SemaphoreType.DMA,
            pltpu.SemaphoreType.DMA,
            pltpu.SemaphoreType.DMA,
            pltpu.SemaphoreType.DMA,
            pltpu.SemaphoreType.DMA,
            pltpu.SemaphoreType.DMA,
            pltpu.SemaphoreType.DMA,
            pltpu.SemaphoreType.DMA,
        ],
    )(_body)
    return k(table_t, idx_flat, tail_1d)


def kernel(table, idxes):
    table_t = jnp.transpose(table)
    idx_flat = jnp.transpose(idxes).reshape(NUM_IDX).astype(jnp.int32)
    tail_1d = jnp.transpose(table[TAIL_B_OFF:, :]).reshape(DIM * TAIL_B)
    out_flat = _gather_t(table_t, idx_flat, tail_1d)
    out3 = out_flat.reshape(DIM, B_COLS, B_ROWS)
    return jnp.transpose(out3, (2, 1, 0))


# speedup vs baseline: 2.5006x; 1.0274x over previous
import functools

import jax
import jax.numpy as jnp
from jax import lax
from jax.experimental import pallas as pl
from jax.experimental.pallas import tpu as pltpu
from jax.experimental.pallas import tpu_sc as plsc

VOCAB = 1000008
DIM = 300
B_ROWS = 4096
B_COLS = 200
NUM_IDX = B_ROWS * B_COLS

NC = 2
NS = 16
C_PER_SC = DIM // NC
J_PER_TILE = NUM_IDX // NS
GCHUNK = 3200
N_G = J_PER_TILE // GCHUNK

CHUNK = 3840
LAST_CHUNK = 1024
N_CHUNKS = 17
PER_TILE_LOAD = CHUNK * 16 + LAST_CHUNK
MAIN_N = NS * PER_TILE_LOAD
TAIL_A = 512
TAIL_B = VOCAB - MAIN_N - TAIL_A
TAIL_B_OFF = MAIN_N + TAIL_A


def _csize(k):
    return CHUNK if k < 16 else LAST_CHUNK


def _body(table_hbm, idx_hbm, tail_hbm, out_hbm,
          img, tail_sp, idx_v, buf0, buf1, buf2, buf3, cbuf, tailrow,
          isem0, isem1, isem2, isem3, fsem,
          gsem0, gsem1, gsem2, gsem3, wsem0, wsem1, wsem2, wsem3):
    sc = lax.axis_index("c")
    tid = lax.axis_index("s")
    cbase = sc * C_PER_SC
    jbase = tid * J_PER_TILE
    lbase = tid * PER_TILE_LOAD

    pltpu.sync_copy(idx_hbm.at[pl.ds(jbase, J_PER_TILE)], idx_v)

    @pl.when(tid == 0)
    def _():
        pltpu.sync_copy(tail_hbm, tail_sp)

    bufs = (buf0, buf1, buf2, buf3)
    isems = (isem0, isem1, isem2, isem3)
    gsems = (gsem0, gsem1, gsem2, gsem3)
    wsems = (wsem0, wsem1, wsem2, wsem3)

    def ind_desc(k):
        cref = cbuf.at[pl.ds(0, 1)]
        n = _csize(k)
        return pltpu.make_async_copy(
            table_hbm.at[cref, pl.ds(lbase + k * CHUNK, n)],
            bufs[k % 4].at[:, pl.ds(0, n)],
            isems[k % 4],
        )

    def fwd_desc(k):
        n = _csize(k)
        return pltpu.make_async_copy(
            bufs[k % 4].at[0, pl.ds(0, n)],
            img.at[pl.ds(lbase + k * CHUNK, n)],
            fsem,
        )

    def indA_desc():
        cref = cbuf.at[pl.ds(0, 1)]
        return pltpu.make_async_copy(
            table_hbm.at[cref, pl.ds(MAIN_N, TAIL_A)],
            buf1.at[:, pl.ds(0, TAIL_A)],
            isems[1],
        )

    tailA_fwd = pltpu.make_async_copy(
        buf1.at[0, pl.ds(0, TAIL_A)], img.at[pl.ds(MAIN_N, TAIL_A)], fsem)
    tailB_fwd = pltpu.make_async_copy(
        tailrow, img.at[pl.ds(TAIL_B_OFF, TAIL_B)], fsem)

    def start_load(c):
        cbuf[...] = jnp.full((16,), c, jnp.int32)
        for q in range(4):
            ind_desc(q).start()
        for k in range(N_CHUNKS):
            ind_desc(k).wait()
            fwd_desc(k).start()
            if k + 4 < N_CHUNKS:
                fwd_desc(k).wait()
                ind_desc(k + 4).start()

        @pl.when(tid == 15)
        def _():
            fwd_desc(13).wait()
            ia = indA_desc()
            ia.start()
            ia.wait()
            tailA_fwd.start()

        @pl.when(tid == 0)
        def _():
            tb = pltpu.make_async_copy(
                tail_sp.at[pl.ds(c * TAIL_B, TAIL_B)], tailrow, isem1)
            tb.start()
            tb.wait()
            tailB_fwd.start()

    def wait_load():
        @pl.when(tid != 15)
        def _():
            fwd_desc(13).wait()
        fwd_desc(14).wait()
        fwd_desc(15).wait()
        fwd_desc(16).wait()

        @pl.when(tid == 15)
        def _():
            tailA_fwd.wait()

        @pl.when(tid == 0)
        def _():
            tailB_fwd.wait()

    def g_desc(b):
        return pltpu.make_async_copy(
            img.at[idx_v.at[pl.ds(b * GCHUNK, GCHUNK)]],
            bufs[b % 4].at[0, pl.ds(0, GCHUNK)],
            gsems[b % 4],
        )

    def w_desc(c, b):
        return pltpu.make_async_copy(
            bufs[b % 4].at[0, pl.ds(0, GCHUNK)],
            out_hbm.at[pl.ds(c * NUM_IDX + jbase + b * GCHUNK, GCHUNK)],
            wsems[b % 4],
        )

    def iter_body(i, carry):
        c = cbase + i
        wait_load()
        plsc.subcore_barrier()
        for q in range(4):
            g_desc(q).start()
        for b in range(N_G):
            g_desc(b).wait()
            w_desc(c, b).start()
            if b + 4 < N_G:
                w_desc(c, b).wait()
                g_desc(b + 4).start()
        for b in range(N_G - 4, N_G):
            w_desc(c, b).wait()
        plsc.subcore_barrier()

        @pl.when(i + 1 < C_PER_SC)
        def _():
            start_load(c + 1)

        return carry

    start_load(cbase)
    lax.fori_loop(0, C_PER_SC, iter_body, 0)


def _gather_t(table_t, idx_flat, tail_1d):
    mesh = plsc.VectorSubcoreMesh(core_axis_name="c", subcore_axis_name="s")
    k = functools.partial(
        pl.kernel,
        mesh=mesh,
        out_type=jax.ShapeDtypeStruct((DIM * NUM_IDX,), jnp.float32),
        scratch_types=[
            pltpu.VMEM_SHARED((VOCAB,), jnp.float32),
            pltpu.VMEM_SHARED((DIM * TAIL_B,), jnp.float32),
            pltpu.VMEM((J_PER_TILE,), jnp.int32),
            pltpu.VMEM((1, CHUNK), jnp.float32),
            pltpu.VMEM((1, CHUNK), jnp.float32),
            pltpu.VMEM((1, CHUNK), jnp.float32),
            pltpu.VMEM((1, CHUNK), jnp.float32),
            pltpu.VMEM((16,), jnp.int32),
            pltpu.VMEM((TAIL_B,), jnp.float32),
            pltpu.SemaphoreType.DMA,
            pltpu.SemaphoreType.DMA,
            pltpu.SemaphoreType.DMA,
            pltpu.SemaphoreType.DMA,
            pltpu.SemaphoreType.DMA,
            pltpu.SemaphoreType.DMA,
            pltpu.SemaphoreType.DMA,
            pltpu.SemaphoreType.DMA,
            pltpu.SemaphoreType.DMA,
            pltpu.SemaphoreType.DMA,
            pltpu.SemaphoreType.DMA,
            pltpu.SemaphoreType.DMA,
            pltpu.SemaphoreType.DMA,
        ],
    )(_body)
    return k(table_t, idx_flat, tail_1d)


def kernel(table, idxes):
    table_t = jnp.transpose(table)
    idx_flat = jnp.transpose(idxes).reshape(NUM_IDX).astype(jnp.int32)
    tail_1d = jnp.transpose(table[TAIL_B_OFF:, :]).reshape(DIM * TAIL_B)
    out_flat = _gather_t(table_t, idx_flat, tail_1d)
    out3 = out_flat.reshape(DIM, B_COLS, B_ROWS)
    return jnp.transpose(out3, (2, 1, 0))
